# Initial kernel scaffold; baseline (speedup 1.0000x reference)
import jax
import jax.numpy as jnp
from jax.experimental import pallas as pl


def kernel(x, W, b, g):
    raise NotImplementedError("write your pallas kernel here")


# speedup vs baseline: 1.8950x; 1.8950x over previous
import functools

import jax
import jax.numpy as jnp
from jax.experimental import pallas as pl

DIM = 2048
NUM_EXPERTS = 64
BLOCK_T = 512


def _gate_kernel(x_ref, w_ref, b_ref, g_ref, o_ref):
    x = x_ref[...]
    g = g_ref[...]
    var = jnp.mean(x * x, axis=-1, keepdims=True)
    xn = x * jax.lax.rsqrt(var + 1e-6) * g[None, :]
    scale = 127.0 / jnp.clip(
        jnp.max(jnp.abs(xn), axis=-1, keepdims=True), 1e-5, None
    )
    xq = jnp.clip(jnp.round(xn * scale), -128.0, 127.0) / scale
    w = w_ref[...]
    ws = 1.0 / jnp.clip(jnp.mean(jnp.abs(w)), 1e-5, None)
    wq = jnp.clip(jnp.round(w * ws), -1.0, 1.0) / ws
    logits = jax.lax.dot_general(
        xq, wq,
        dimension_numbers=(((1,), (1,)), ((), ())),
        preferred_element_type=jnp.float32,
    ) + b_ref[...][None, :]
    m = jnp.max(logits, axis=-1, keepdims=True)
    e = jnp.exp(logits - m)
    o_ref[...] = e / jnp.sum(e, axis=-1, keepdims=True)


@jax.jit
def kernel(x, W, b, g):
    tokens = x.shape[0]
    grid = (tokens // BLOCK_T,)
    return pl.pallas_call(
        _gate_kernel,
        grid=grid,
        in_specs=[
            pl.BlockSpec((BLOCK_T, DIM), lambda i: (i, 0)),
            pl.BlockSpec((NUM_EXPERTS, DIM), lambda i: (0, 0)),
            pl.BlockSpec((NUM_EXPERTS,), lambda i: (0,)),
            pl.BlockSpec((DIM,), lambda i: (0,)),
        ],
        out_specs=pl.BlockSpec((BLOCK_T, NUM_EXPERTS), lambda i: (i, 0)),
        out_shape=jax.ShapeDtypeStruct((tokens, NUM_EXPERTS), jnp.float32),
    )(x, W, b, g)
